# Initial kernel scaffold; baseline (speedup 1.0000x reference)
import jax
import jax.numpy as jnp
from jax.experimental import pallas as pl


def kernel(x, table, fc_w, fc_b):
    raise NotImplementedError("write your pallas kernel here")


# speedup vs baseline: 11.9673x; 11.9673x over previous
import functools

import jax
import jax.numpy as jnp
from jax import lax
from jax.experimental import pallas as pl
from jax.experimental.pallas import tpu as pltpu
from jax.experimental.pallas import tpu_sc as plsc

VOCAB = 1000000
EMBED = 32
NUM_CLASSES = 4
BATCH = 16384
SEQ = 200

_INFO = plsc.get_sparse_core_info()
NC = _INFO.num_cores
NS = _INFO.num_subcores
NW = NC * NS
ROWS_PER_W = BATCH // NW
CHUNK = 4
N_CHUNKS = ROWS_PER_W // CHUNK
HALF_SEQ = SEQ // 2
G_PER_CHUNK = 2 * CHUNK


def _sc_pooled_classify(x2, table, fw_pat, bias_pat):
    mesh = plsc.VectorSubcoreMesh(core_axis_name="c", subcore_axis_name="s")

    @functools.partial(
        pl.kernel,
        out_type=jax.ShapeDtypeStruct((BATCH // CHUNK, 16), jnp.float32),
        mesh=mesh,
        scratch_types=[
            pltpu.VMEM((G_PER_CHUNK, HALF_SEQ), jnp.int32),
            pltpu.VMEM((CHUNK * SEQ, EMBED), jnp.float32),
            pltpu.VMEM((N_CHUNKS, 16), jnp.float32),
            pltpu.VMEM((EMBED, 16), jnp.float32),
            pltpu.VMEM((16,), jnp.float32),
            pltpu.VMEM((CHUNK * EMBED,), jnp.float32),
            pltpu.SemaphoreType.DMA,
        ],
        compiler_params=pltpu.CompilerParams(
            needs_layout_passes=False, use_tc_tiling_on_sc=False),
    )
    def k(x2_hbm, tbl_hbm, fwp_hbm, fcb_hbm, out_hbm, xbuf, rows_v, outbuf,
          fwp_v, fcb_v, acc_v, sem):
        wid = lax.axis_index("s") * NC + lax.axis_index("c")
        row0 = wid * ROWS_PER_W

        pltpu.sync_copy(fwp_hbm, fwp_v)
        pltpu.sync_copy(fcb_hbm, fcb_v)
        inv_s = jnp.float32(1.0 / SEQ)
        bias_v = fcb_v[pl.ds(0, 16)]
        zero = jnp.zeros((16,), jnp.float32)
        lane = lax.iota(jnp.int32, 16)
        rbase = (lane >> 2) * EMBED

        def chunk_body(ci, _):
            b0 = row0 + ci * CHUNK
            pltpu.sync_copy(x2_hbm.at[pl.ds(b0 * 2, G_PER_CHUNK), :], xbuf)
            cps = [
                pltpu.async_copy(
                    tbl_hbm.at[xbuf.at[j]],
                    rows_v.at[pl.ds(j * HALF_SEQ, HALF_SEQ), :],
                    sem,
                )
                for j in range(G_PER_CHUNK)
            ]
            for cp in cps:
                cp.wait()

            for r in range(CHUNK):
                def inner(iv, carry, r=r):
                    a0, a1 = carry
                    base = r * SEQ + iv * 8
                    for u in range(8):
                        a0 = a0 + rows_v[base + u, pl.ds(0, 16)]
                        a1 = a1 + rows_v[base + u, pl.ds(16, 16)]
                    return a0, a1

                a0, a1 = lax.fori_loop(0, SEQ // 8, inner, (zero, zero))
                acc_v[pl.ds(r * EMBED, 16)] = a0
                acc_v[pl.ds(r * EMBED + 16, 16)] = a1

            res = zero
            for d in range(EMBED):
                a_d = plsc.load_gather(acc_v, [rbase + d])
                res = res + a_d * fwp_v[d, pl.ds(0, 16)]
            outbuf[ci, pl.ds(0, 16)] = res * inv_s + bias_v
            return ()

        lax.fori_loop(0, N_CHUNKS, chunk_body, ())
        pltpu.sync_copy(outbuf, out_hbm.at[pl.ds(wid * N_CHUNKS, N_CHUNKS), :])

    return k(x2, table, fw_pat, bias_pat)


def kernel(x, table, fc_w, fc_b):
    x2 = x.astype(jnp.int32).reshape(2 * BATCH, HALF_SEQ)
    fw_pat = jnp.tile(fc_w.T.astype(jnp.float32), (1, CHUNK))
    bias_pat = jnp.tile(fc_b.astype(jnp.float32), CHUNK)
    out = _sc_pooled_classify(x2, table, fw_pat, bias_pat)
    return out.reshape(BATCH, NUM_CLASSES)


# speedup vs baseline: 15.5448x; 1.2989x over previous
import functools

import jax
import jax.numpy as jnp
from jax import lax
from jax.experimental import pallas as pl
from jax.experimental.pallas import tpu as pltpu
from jax.experimental.pallas import tpu_sc as plsc

VOCAB = 1000000
EMBED = 32
NUM_CLASSES = 4
BATCH = 16384
SEQ = 200

_INFO = plsc.get_sparse_core_info()
NC = _INFO.num_cores
NS = _INFO.num_subcores
NW = NC * NS
ROWS_PER_W = BATCH // NW
CHUNK = 8
N_CHUNKS = ROWS_PER_W // CHUNK
HALF_SEQ = SEQ // 2
G_PER_CHUNK = 2 * CHUNK
VECS_PER_CHUNK = CHUNK // 4


def _sc_pooled_classify(x2, table, fw_pat, bias_pat):
    mesh = plsc.VectorSubcoreMesh(core_axis_name="c", subcore_axis_name="s")

    @functools.partial(
        pl.kernel,
        out_type=jax.ShapeDtypeStruct((BATCH // 4, 16), jnp.float32),
        mesh=mesh,
        scratch_types=[
            pltpu.VMEM((2, G_PER_CHUNK, HALF_SEQ), jnp.int32),
            pltpu.VMEM((2, CHUNK * SEQ, EMBED), jnp.float32),
            pltpu.VMEM((N_CHUNKS * VECS_PER_CHUNK, 16), jnp.float32),
            pltpu.VMEM((EMBED, 16), jnp.float32),
            pltpu.VMEM((16,), jnp.float32),
            pltpu.VMEM((CHUNK * EMBED,), jnp.float32),
            pltpu.SemaphoreType.DMA,
            pltpu.SemaphoreType.DMA,
        ],
        compiler_params=pltpu.CompilerParams(
            needs_layout_passes=False, use_tc_tiling_on_sc=False),
    )
    def k(x2_hbm, tbl_hbm, fwp_hbm, fcb_hbm, out_hbm, xbuf, rows_v, outbuf,
          fwp_v, fcb_v, acc_v, sem0, sem1):
        wid = lax.axis_index("s") * NC + lax.axis_index("c")
        row0 = wid * ROWS_PER_W
        sems = (sem0, sem1)

        pltpu.sync_copy(fwp_hbm, fwp_v)
        pltpu.sync_copy(fcb_hbm, fcb_v)
        inv_s = jnp.float32(1.0 / SEQ)
        bias_v = fcb_v[pl.ds(0, 16)]
        zero = jnp.zeros((16,), jnp.float32)
        lane = lax.iota(jnp.int32, 16)
        rbase = (lane >> 2) * EMBED

        def issue(ci, b):
            b0 = row0 + ci * CHUNK
            pltpu.sync_copy(x2_hbm.at[pl.ds(b0 * 2, G_PER_CHUNK), :],
                            xbuf.at[b])
            for j in range(G_PER_CHUNK):
                pltpu.async_copy(
                    tbl_hbm.at[xbuf.at[b, j]],
                    rows_v.at[b, pl.ds(j * HALF_SEQ, HALF_SEQ), :],
                    sems[b],
                )

        def wait_buf(b):
            pltpu.make_async_copy(
                tbl_hbm.at[pl.ds(0, CHUNK * SEQ), :], rows_v.at[b], sems[b]
            ).wait()

        def compute(ci, b):
            for r in range(CHUNK):
                def inner(s, carry, r=r):
                    a0, a1 = carry
                    i = r * SEQ + s
                    return (a0 + rows_v[b, i, pl.ds(0, 16)],
                            a1 + rows_v[b, i, pl.ds(16, 16)])

                a0, a1 = lax.fori_loop(0, SEQ, inner, (zero, zero), unroll=8)
                acc_v[pl.ds(r * EMBED, 16)] = a0
                acc_v[pl.ds(r * EMBED + 16, 16)] = a1

            for g in range(VECS_PER_CHUNK):
                res = zero
                idx0 = rbase + g * 4 * EMBED
                for d in range(EMBED):
                    a_d = plsc.load_gather(acc_v, [idx0 + d])
                    res = res + a_d * fwp_v[d, pl.ds(0, 16)]
                outbuf[ci * VECS_PER_CHUNK + g, pl.ds(0, 16)] = (
                    res * inv_s + bias_v)

        issue(0, 0)

        def body2(h, _):
            for p in range(2):
                ci = h * 2 + p
                nxt = lax.rem(ci + 1, N_CHUNKS)
                issue(nxt, 1 - p)
                wait_buf(p)
                compute(ci, p)
            return ()

        lax.fori_loop(0, N_CHUNKS // 2, body2, ())
        wait_buf(0)

        n_out = N_CHUNKS * VECS_PER_CHUNK
        pltpu.sync_copy(outbuf, out_hbm.at[pl.ds(wid * n_out, n_out), :])

    return k(x2, table, fw_pat, bias_pat)


def kernel(x, table, fc_w, fc_b):
    x2 = x.astype(jnp.int32).reshape(2 * BATCH, HALF_SEQ)
    fw_pat = jnp.tile(fc_w.T.astype(jnp.float32), (1, 4))
    bias_pat = jnp.tile(fc_b.astype(jnp.float32), 4)
    out = _sc_pooled_classify(x2, table, fw_pat, bias_pat)
    return out.reshape(BATCH, NUM_CLASSES)


# speedup vs baseline: 15.6211x; 1.0049x over previous
import functools

import jax
import jax.numpy as jnp
from jax import lax
from jax.experimental import pallas as pl
from jax.experimental.pallas import tpu as pltpu
from jax.experimental.pallas import tpu_sc as plsc

VOCAB = 1000000
EMBED = 32
NUM_CLASSES = 4
BATCH = 16384
SEQ = 200

_INFO = plsc.get_sparse_core_info()
NC = _INFO.num_cores
NS = _INFO.num_subcores
NW = NC * NS
ROWS_PER_W = BATCH // NW
CHUNK = 8
N_CHUNKS = ROWS_PER_W // CHUNK
HALF_SEQ = SEQ // 2
G_PER_CHUNK = 2 * CHUNK
VECS_PER_CHUNK = CHUNK // 4


def _sc_pooled_classify(x2, table_pad, fw_pat, bias_pat):
    mesh = plsc.VectorSubcoreMesh(core_axis_name="c", subcore_axis_name="s")

    @functools.partial(
        pl.kernel,
        out_type=jax.ShapeDtypeStruct((BATCH // 4, 16), jnp.float32),
        mesh=mesh,
        scratch_types=[
            pltpu.VMEM((2, G_PER_CHUNK, HALF_SEQ), jnp.int32),
            pltpu.VMEM((2, CHUNK * SEQ, EMBED), jnp.float32),
            pltpu.VMEM((N_CHUNKS * VECS_PER_CHUNK, 16), jnp.float32),
            pltpu.VMEM((EMBED, 16), jnp.float32),
            pltpu.VMEM((16,), jnp.float32),
            pltpu.VMEM((CHUNK * EMBED,), jnp.float32),
            pltpu.SemaphoreType.DMA,
            pltpu.SemaphoreType.DMA,
        ],
        compiler_params=pltpu.CompilerParams(
            needs_layout_passes=False, use_tc_tiling_on_sc=False),
    )
    def k(x2_hbm, tbl_hbm, fwp_hbm, fcb_hbm, out_hbm, xbuf, rows_v, outbuf,
          fwp_v, fcb_v, acc_v, sem0, sem1):
        wid = lax.axis_index("s") * NC + lax.axis_index("c")
        row0 = wid * ROWS_PER_W
        sems = (sem0, sem1)
        tbl4 = tbl_hbm

        pltpu.sync_copy(fwp_hbm, fwp_v)
        pltpu.sync_copy(fcb_hbm, fcb_v)
        inv_s = jnp.float32(1.0 / SEQ)
        bias_v = fcb_v[pl.ds(0, 16)]
        zero = jnp.zeros((16,), jnp.float32)
        lane = lax.iota(jnp.int32, 16)
        rbase = (lane >> 2) * EMBED

        def issue(ci, b):
            b0 = row0 + ci * CHUNK
            pltpu.sync_copy(x2_hbm.at[pl.ds(b0 * 2, G_PER_CHUNK), :],
                            xbuf.at[b])
            for j in range(G_PER_CHUNK):
                pltpu.async_copy(
                    tbl4.at[xbuf.at[b, j]],
                    rows_v.at[b, pl.ds(j * HALF_SEQ, HALF_SEQ), :],
                    sems[b],
                )

        def wait_buf(b):
            pltpu.make_async_copy(
                tbl4.at[pl.ds(0, CHUNK * SEQ), :], rows_v.at[b], sems[b]
            ).wait()

        def compute(ci, b):
            for r in range(CHUNK):
                def inner(s, carry, r=r):
                    a0, a1 = carry
                    i = r * SEQ + s
                    return (a0 + rows_v[b, i, pl.ds(0, 16)],
                            a1 + rows_v[b, i, pl.ds(16, 16)])

                a0, a1 = lax.fori_loop(0, SEQ, inner, (zero, zero), unroll=8)
                acc_v[pl.ds(r * EMBED, 16)] = a0
                acc_v[pl.ds(r * EMBED + 16, 16)] = a1

            for g in range(VECS_PER_CHUNK):
                res = zero
                idx0 = rbase + g * 4 * EMBED
                for d in range(EMBED):
                    a_d = plsc.load_gather(acc_v, [idx0 + d])
                    res = res + a_d * fwp_v[d, pl.ds(0, 16)]
                outbuf[ci * VECS_PER_CHUNK + g, pl.ds(0, 16)] = (
                    res * inv_s + bias_v)

        issue(0, 0)

        def body2(h, _):
            for p in range(2):
                ci = h * 2 + p
                nxt = lax.rem(ci + 1, N_CHUNKS)
                issue(nxt, 1 - p)
                wait_buf(p)
                compute(ci, p)
            return ()

        lax.fori_loop(0, N_CHUNKS // 2, body2, ())
        wait_buf(0)

        n_out = N_CHUNKS * VECS_PER_CHUNK
        pltpu.sync_copy(outbuf, out_hbm.at[pl.ds(wid * n_out, n_out), :])

    return k(x2, table_pad, fw_pat, bias_pat)


def kernel(x, table, fc_w, fc_b):
    x2 = (x.astype(jnp.int32) * 4).reshape(2 * BATCH, HALF_SEQ)
    table_pad = jnp.pad(table, ((0, 0), (0, 128 - EMBED)))
    table_pad = table_pad.reshape(4 * VOCAB, EMBED)
    fw_pat = jnp.tile(fc_w.T.astype(jnp.float32), (1, 4))
    bias_pat = jnp.tile(fc_b.astype(jnp.float32), 4)
    out = _sc_pooled_classify(x2, table_pad, fw_pat, bias_pat)
    return out.reshape(BATCH, NUM_CLASSES)


# speedup vs baseline: 16.5404x; 1.0588x over previous
import functools

import jax
import jax.numpy as jnp
from jax import lax
from jax.experimental import pallas as pl
from jax.experimental.pallas import tpu as pltpu
from jax.experimental.pallas import tpu_sc as plsc

VOCAB = 1000000
EMBED = 32
NUM_CLASSES = 4
BATCH = 16384
SEQ = 200

_INFO = plsc.get_sparse_core_info()
NC = _INFO.num_cores
NS = _INFO.num_subcores
NW = NC * NS
ROWS_PER_W = BATCH // NW
CHUNK = 8
N_CHUNKS = ROWS_PER_W // CHUNK
HALF_SEQ = SEQ // 2
G_PER_CHUNK = 2 * CHUNK
VECS_PER_CHUNK = CHUNK // 4

_PACK_BV = 8192


def _tc_pack_bf16(tbl_t):
    def body(tin, tout):
        blk = tin[...]
        eye = jnp.eye(EMBED, dtype=jnp.float32)
        tr = lax.dot_general(blk, eye, (((0,), (0,)), ((), ())),
                             preferred_element_type=jnp.float32)
        lo = lax.bitcast_convert_type(
            tr[:, 0:16].astype(jnp.bfloat16), jnp.uint16).astype(jnp.uint32)
        hi = lax.bitcast_convert_type(
            tr[:, 16:EMBED].astype(jnp.bfloat16), jnp.uint16).astype(jnp.uint32)
        w = (hi << 16) | lo
        tout[:, 0:16] = lax.bitcast_convert_type(w, jnp.float32)

    return pl.pallas_call(
        body,
        grid=(VOCAB // _PACK_BV,),
        in_specs=[pl.BlockSpec((EMBED, _PACK_BV), lambda i: (0, i))],
        out_specs=pl.BlockSpec((_PACK_BV, 128), lambda i: (i, 0)),
        out_shape=jax.ShapeDtypeStruct((VOCAB, 128), jnp.float32),
    )(tbl_t)


def _sc_pooled_classify(x2, table8, fw_pat, bias_pat):
    mesh = plsc.VectorSubcoreMesh(core_axis_name="c", subcore_axis_name="s")

    @functools.partial(
        pl.kernel,
        out_type=jax.ShapeDtypeStruct((BATCH // 4, 16), jnp.float32),
        mesh=mesh,
        scratch_types=[
            pltpu.VMEM((2, G_PER_CHUNK, HALF_SEQ), jnp.int32),
            pltpu.VMEM((2, CHUNK * SEQ, 16), jnp.float32),
            pltpu.VMEM((N_CHUNKS * VECS_PER_CHUNK, 16), jnp.float32),
            pltpu.VMEM((EMBED, 16), jnp.float32),
            pltpu.VMEM((16,), jnp.float32),
            pltpu.VMEM((CHUNK * EMBED,), jnp.float32),
            pltpu.SemaphoreType.DMA,
            pltpu.SemaphoreType.DMA,
        ],
        compiler_params=pltpu.CompilerParams(
            needs_layout_passes=False, use_tc_tiling_on_sc=False),
    )
    def k(x2_hbm, tbl_hbm, fwp_hbm, fcb_hbm, out_hbm, xbuf, rows_v, outbuf,
          fwp_v, fcb_v, acc_v, sem0, sem1):
        wid = lax.axis_index("s") * NC + lax.axis_index("c")
        row0 = wid * ROWS_PER_W
        sems = (sem0, sem1)

        pltpu.sync_copy(fwp_hbm, fwp_v)
        pltpu.sync_copy(fcb_hbm, fcb_v)
        inv_s = jnp.float32(1.0 / SEQ)
        bias_v = fcb_v[pl.ds(0, 16)]
        zero = jnp.zeros((16,), jnp.float32)
        lane = lax.iota(jnp.int32, 16)
        rbase = (lane >> 2) * EMBED
        himask = jnp.int32(-65536)

        def issue(ci, b):
            b0 = row0 + ci * CHUNK
            pltpu.sync_copy(x2_hbm.at[pl.ds(b0 * 2, G_PER_CHUNK), :],
                            xbuf.at[b])
            for j in range(G_PER_CHUNK):
                pltpu.async_copy(
                    tbl_hbm.at[xbuf.at[b, j]],
                    rows_v.at[b, pl.ds(j * HALF_SEQ, HALF_SEQ), :],
                    sems[b],
                )

        def wait_buf(b):
            pltpu.make_async_copy(
                tbl_hbm.at[pl.ds(0, CHUNK * SEQ), :], rows_v.at[b], sems[b]
            ).wait()

        def compute(ci, b):
            for r in range(CHUNK):
                def inner(s, carry, r=r):
                    a0, a1 = carry
                    w = plsc.bitcast(rows_v[b, r * SEQ + s, pl.ds(0, 16)],
                                     jnp.int32)
                    lo = plsc.bitcast(w << 16, jnp.float32)
                    hi = plsc.bitcast(w & himask, jnp.float32)
                    return a0 + lo, a1 + hi

                a0, a1 = lax.fori_loop(0, SEQ, inner, (zero, zero), unroll=8)
                acc_v[pl.ds(r * EMBED, 16)] = a0
                acc_v[pl.ds(r * EMBED + 16, 16)] = a1

            for g in range(VECS_PER_CHUNK):
                res = zero
                idx0 = rbase + g * 4 * EMBED
                for d in range(EMBED):
                    a_d = plsc.load_gather(acc_v, [idx0 + d])
                    res = res + a_d * fwp_v[d, pl.ds(0, 16)]
                outbuf[ci * VECS_PER_CHUNK + g, pl.ds(0, 16)] = (
                    res * inv_s + bias_v)

        issue(0, 0)

        def body2(h, _):
            for p in range(2):
                ci = h * 2 + p
                nxt = lax.rem(ci + 1, N_CHUNKS)
                issue(nxt, 1 - p)
                wait_buf(p)
                compute(ci, p)
            return ()

        lax.fori_loop(0, N_CHUNKS // 2, body2, ())
        wait_buf(0)

        n_out = N_CHUNKS * VECS_PER_CHUNK
        pltpu.sync_copy(outbuf, out_hbm.at[pl.ds(wid * n_out, n_out), :])

    return k(x2, table8, fw_pat, bias_pat)


def kernel(x, table, fc_w, fc_b):
    x2 = (x.astype(jnp.int32) * 8).reshape(2 * BATCH, HALF_SEQ)
    tbl_pk = _tc_pack_bf16(table.T)
    table8 = tbl_pk.reshape(8 * VOCAB, 16)
    fw_pat = jnp.tile(fc_w.T.astype(jnp.float32), (1, 4))
    bias_pat = jnp.tile(fc_b.astype(jnp.float32), 4)
    out = _sc_pooled_classify(x2, table8, fw_pat, bias_pat)
    return out.reshape(BATCH, NUM_CLASSES)


# speedup vs baseline: 21.7845x; 1.3171x over previous
import functools

import jax
import jax.numpy as jnp
from jax import lax
from jax.experimental import pallas as pl
from jax.experimental.pallas import tpu as pltpu
from jax.experimental.pallas import tpu_sc as plsc

VOCAB = 1000000
EMBED = 32
NUM_CLASSES = 4
BATCH = 16384
SEQ = 200

_INFO = plsc.get_sparse_core_info()
NC = _INFO.num_cores
NS = _INFO.num_subcores
NW = NC * NS
ROWS_PER_W = BATCH // NW
CHUNK = 8
N_CHUNKS = ROWS_PER_W // CHUNK
HALF_SEQ = SEQ // 2
G_PER_CHUNK = 2 * CHUNK
VECS_PER_CHUNK = CHUNK // 4

_PACK_BV = 8192


def _tc_pack_bf16(tbl_t):
    def body(tin, tout):
        blk = tin[...]
        u = lax.bitcast_convert_type(blk, jnp.uint32)
        r = (u + jnp.uint32(0x7FFF) + ((u >> 16) & jnp.uint32(1))) \
            & jnp.uint32(0xFFFF0000)
        w = (r[0:16, :] >> 16) | r[16:EMBED, :]
        wf = lax.bitcast_convert_type(w, jnp.float32)
        tout[:, 0:16] = lax.transpose(wf, (1, 0))

    return pl.pallas_call(
        body,
        grid=(VOCAB // _PACK_BV,),
        in_specs=[pl.BlockSpec((EMBED, _PACK_BV), lambda i: (0, i))],
        out_specs=pl.BlockSpec((_PACK_BV, 128), lambda i: (i, 0)),
        out_shape=jax.ShapeDtypeStruct((VOCAB, 128), jnp.float32),
    )(tbl_t)


def _sc_pooled_classify(x2, table8, fw_pat, bias_pat):
    mesh = plsc.VectorSubcoreMesh(core_axis_name="c", subcore_axis_name="s")

    @functools.partial(
        pl.kernel,
        out_type=jax.ShapeDtypeStruct((BATCH // 4, 16), jnp.float32),
        mesh=mesh,
        scratch_types=[
            pltpu.VMEM((2, G_PER_CHUNK, HALF_SEQ), jnp.int32),
            pltpu.VMEM((2, CHUNK * SEQ, 16), jnp.float32),
            pltpu.VMEM((N_CHUNKS * VECS_PER_CHUNK, 16), jnp.float32),
            pltpu.VMEM((EMBED, 16), jnp.float32),
            pltpu.VMEM((16,), jnp.float32),
            pltpu.VMEM((CHUNK * EMBED,), jnp.float32),
            pltpu.SemaphoreType.DMA,
            pltpu.SemaphoreType.DMA,
        ],
        compiler_params=pltpu.CompilerParams(
            needs_layout_passes=False, use_tc_tiling_on_sc=False),
    )
    def k(x2_hbm, tbl_hbm, fwp_hbm, fcb_hbm, out_hbm, xbuf, rows_v, outbuf,
          fwp_v, fcb_v, acc_v, sem0, sem1):
        wid = lax.axis_index("s") * NC + lax.axis_index("c")
        row0 = wid * ROWS_PER_W
        sems = (sem0, sem1)

        pltpu.sync_copy(fwp_hbm, fwp_v)
        pltpu.sync_copy(fcb_hbm, fcb_v)
        inv_s = jnp.float32(1.0 / SEQ)
        bias_v = fcb_v[pl.ds(0, 16)]
        zero = jnp.zeros((16,), jnp.float32)
        lane = lax.iota(jnp.int32, 16)
        rbase = (lane >> 2) * EMBED
        himask = jnp.int32(-65536)

        def issue(ci, b):
            b0 = row0 + ci * CHUNK
            pltpu.sync_copy(x2_hbm.at[pl.ds(b0 * 2, G_PER_CHUNK), :],
                            xbuf.at[b])
            for j in range(G_PER_CHUNK):
                pltpu.async_copy(
                    tbl_hbm.at[xbuf.at[b, j]],
                    rows_v.at[b, pl.ds(j * HALF_SEQ, HALF_SEQ), :],
                    sems[b],
                )

        def wait_buf(b):
            pltpu.make_async_copy(
                tbl_hbm.at[pl.ds(0, CHUNK * SEQ), :], rows_v.at[b], sems[b]
            ).wait()

        def compute(ci, b):
            for r in range(CHUNK):
                def inner(s, carry, r=r):
                    a0, a1 = carry
                    w = plsc.bitcast(rows_v[b, r * SEQ + s, pl.ds(0, 16)],
                                     jnp.int32)
                    lo = plsc.bitcast(w << 16, jnp.float32)
                    hi = plsc.bitcast(w & himask, jnp.float32)
                    return a0 + lo, a1 + hi

                a0, a1 = lax.fori_loop(0, SEQ, inner, (zero, zero), unroll=8)
                acc_v[pl.ds(r * EMBED, 16)] = a0
                acc_v[pl.ds(r * EMBED + 16, 16)] = a1

            for g in range(VECS_PER_CHUNK):
                res = zero
                idx0 = rbase + g * 4 * EMBED
                for d in range(EMBED):
                    a_d = plsc.load_gather(acc_v, [idx0 + d])
                    res = res + a_d * fwp_v[d, pl.ds(0, 16)]
                outbuf[ci * VECS_PER_CHUNK + g, pl.ds(0, 16)] = (
                    res * inv_s + bias_v)

        issue(0, 0)

        def body2(h, _):
            for p in range(2):
                ci = h * 2 + p
                nxt = lax.rem(ci + 1, N_CHUNKS)
                issue(nxt, 1 - p)
                wait_buf(p)
                compute(ci, p)
            return ()

        lax.fori_loop(0, N_CHUNKS // 2, body2, ())
        wait_buf(0)

        n_out = N_CHUNKS * VECS_PER_CHUNK
        pltpu.sync_copy(outbuf, out_hbm.at[pl.ds(wid * n_out, n_out), :])

    return k(x2, table8, fw_pat, bias_pat)


def kernel(x, table, fc_w, fc_b):
    x2 = (x.astype(jnp.int32) * 8).reshape(2 * BATCH, HALF_SEQ)
    tbl_pk = _tc_pack_bf16(table.T)
    table8 = tbl_pk.reshape(8 * VOCAB, 16)
    fw_pat = jnp.tile(fc_w.T.astype(jnp.float32), (1, 4))
    bias_pat = jnp.tile(fc_b.astype(jnp.float32), 4)
    out = _sc_pooled_classify(x2, table8, fw_pat, bias_pat)
    return out.reshape(BATCH, NUM_CLASSES)


# speedup vs baseline: 30.9519x; 1.4208x over previous
import functools

import jax
import jax.numpy as jnp
from jax import lax
from jax.experimental import pallas as pl
from jax.experimental.pallas import tpu as pltpu
from jax.experimental.pallas import tpu_sc as plsc

VOCAB = 1000000
EMBED = 32
NUM_CLASSES = 4
BATCH = 16384
SEQ = 200

_INFO = plsc.get_sparse_core_info()
NC = _INFO.num_cores
NS = _INFO.num_subcores
NW = NC * NS
ROWS_PER_W = BATCH // NW
CHUNK = 16
N_CHUNKS = ROWS_PER_W // CHUNK
IDX_PER_CHUNK = CHUNK * SEQ // 128
VECS_PER_CHUNK = CHUNK // 4

_PACK_BV = 8192
_PACK_BA = _PACK_BV // 8
_PACK_GRID = -(-VOCAB // _PACK_BV)
_PACK_ROWS = _PACK_GRID * _PACK_BA


def _tc_pack_bf16(tbl_t):
    def body(tin, tout):
        blk = tin[...]
        u = lax.bitcast_convert_type(blk, jnp.uint32)
        r = (u + jnp.uint32(0x7FFF) + ((u >> 16) & jnp.uint32(1))) \
            & jnp.uint32(0xFFFF0000)
        w = (r[0:16, :] >> 16) | r[16:EMBED, :]
        w128 = jnp.concatenate(
            [w[:, q * _PACK_BA:(q + 1) * _PACK_BA] for q in range(8)],
            axis=0)
        wf = lax.bitcast_convert_type(w128, jnp.float32)
        tout[...] = lax.transpose(wf, (1, 0))

    return pl.pallas_call(
        body,
        grid=(_PACK_GRID,),
        in_specs=[pl.BlockSpec((EMBED, _PACK_BV), lambda i: (0, i))],
        out_specs=pl.BlockSpec((_PACK_BA, 128), lambda i: (i, 0)),
        out_shape=jax.ShapeDtypeStruct((_PACK_ROWS, 128), jnp.float32),
    )(tbl_t)


def _sc_pooled_classify(x2, table16, fw_pat, bias_pat):
    mesh = plsc.VectorSubcoreMesh(core_axis_name="c", subcore_axis_name="s")

    @functools.partial(
        pl.kernel,
        out_type=jax.ShapeDtypeStruct((BATCH // 4, 16), jnp.float32),
        mesh=mesh,
        scratch_types=[
            pltpu.VMEM((2, IDX_PER_CHUNK, 128), jnp.int32),
            pltpu.VMEM((2, CHUNK * SEQ, 16), jnp.float32),
            pltpu.VMEM((N_CHUNKS * VECS_PER_CHUNK, 16), jnp.float32),
            pltpu.VMEM((EMBED, 16), jnp.float32),
            pltpu.VMEM((16,), jnp.float32),
            pltpu.VMEM((CHUNK * EMBED,), jnp.float32),
            pltpu.SemaphoreType.DMA,
            pltpu.SemaphoreType.DMA,
        ],
        compiler_params=pltpu.CompilerParams(
            needs_layout_passes=False, use_tc_tiling_on_sc=False),
    )
    def k(x2_hbm, tbl_hbm, fwp_hbm, fcb_hbm, out_hbm, xbuf, rows_v, outbuf,
          fwp_v, fcb_v, acc_v, sem0, sem1):
        wid = lax.axis_index("s") * NC + lax.axis_index("c")
        row0 = wid * ROWS_PER_W
        ixrow0 = wid * ROWS_PER_W * SEQ // 128
        sems = (sem0, sem1)

        pltpu.sync_copy(fwp_hbm, fwp_v)
        pltpu.sync_copy(fcb_hbm, fcb_v)
        inv_s = jnp.float32(1.0 / SEQ)
        bias_v = fcb_v[pl.ds(0, 16)]
        zero = jnp.zeros((16,), jnp.float32)
        lane = lax.iota(jnp.int32, 16)
        rbase = (lane >> 2) * EMBED
        himask = jnp.int32(-65536)

        def issue(ci, b):
            pltpu.sync_copy(
                x2_hbm.at[pl.ds(ixrow0 + ci * IDX_PER_CHUNK, IDX_PER_CHUNK), :],
                xbuf.at[b])
            for j in range(IDX_PER_CHUNK):
                pltpu.async_copy(
                    tbl_hbm.at[xbuf.at[b, j]],
                    rows_v.at[b, pl.ds(j * 128, 128), :],
                    sems[b],
                )

        def wait_buf(b):
            pltpu.make_async_copy(
                tbl_hbm.at[pl.ds(0, CHUNK * SEQ), :], rows_v.at[b], sems[b]
            ).wait()

        def compute(ci, b):
            for r in range(CHUNK):
                def inner(s, carry, r=r):
                    a0, a1 = carry
                    w = plsc.bitcast(rows_v[b, r * SEQ + s, pl.ds(0, 16)],
                                     jnp.int32)
                    lo = plsc.bitcast(w << 16, jnp.float32)
                    hi = plsc.bitcast(w & himask, jnp.float32)
                    return a0 + lo, a1 + hi

                a0, a1 = lax.fori_loop(0, SEQ, inner, (zero, zero), unroll=8)
                acc_v[pl.ds(r * EMBED, 16)] = a0
                acc_v[pl.ds(r * EMBED + 16, 16)] = a1

            for g in range(VECS_PER_CHUNK):
                res = zero
                idx0 = rbase + g * 4 * EMBED
                for d in range(EMBED):
                    a_d = plsc.load_gather(acc_v, [idx0 + d])
                    res = res + a_d * fwp_v[d, pl.ds(0, 16)]
                outbuf[ci * VECS_PER_CHUNK + g, pl.ds(0, 16)] = (
                    res * inv_s + bias_v)

        issue(0, 0)

        def body2(h, _):
            for p in range(2):
                ci = h * 2 + p
                nxt = lax.rem(ci + 1, N_CHUNKS)
                issue(nxt, 1 - p)
                wait_buf(p)
                compute(ci, p)
            return ()

        lax.fori_loop(0, N_CHUNKS // 2, body2, ())
        wait_buf(0)

        n_out = N_CHUNKS * VECS_PER_CHUNK
        pltpu.sync_copy(outbuf, out_hbm.at[pl.ds(wid * n_out, n_out), :])

    return k(x2, table16, fw_pat, bias_pat)


def kernel(x, table, fc_w, fc_b):
    y = x.astype(jnp.int32)
    x2 = ((y & jnp.int32(-8192)) | ((y & 1023) << 3) | ((y >> 10) & 7))
    x2 = x2.reshape(BATCH * SEQ // 128, 128)
    tbl_pk = _tc_pack_bf16(table.T)
    table16 = tbl_pk.reshape(_PACK_ROWS * 8, 16)
    fw_pat = jnp.tile(fc_w.T.astype(jnp.float32), (1, 4))
    bias_pat = jnp.tile(fc_b.astype(jnp.float32), 4)
    out = _sc_pooled_classify(x2, table16, fw_pat, bias_pat)
    return out.reshape(BATCH, NUM_CLASSES)


# speedup vs baseline: 34.1933x; 1.1047x over previous
import functools

import jax
import jax.numpy as jnp
from jax import lax
from jax.experimental import pallas as pl
from jax.experimental.pallas import tpu as pltpu
from jax.experimental.pallas import tpu_sc as plsc

VOCAB = 1000000
EMBED = 32
NUM_CLASSES = 4
BATCH = 16384
SEQ = 200

_INFO = plsc.get_sparse_core_info()
NC = _INFO.num_cores
NS = _INFO.num_subcores
NW = NC * NS
ROWS_PER_W = BATCH // NW
CHUNK = 16
N_CHUNKS = ROWS_PER_W // CHUNK
IDX_PER_CHUNK = CHUNK * SEQ // 128
VECS_PER_CHUNK = CHUNK // 4

_PACK_BV = 8192
_PACK_BA = _PACK_BV // 8
_PACK_GRID = -(-VOCAB // _PACK_BV)
_PACK_ROWS = _PACK_GRID * _PACK_BA


def _tc_pack_bf16(tbl_t):
    def body(tin, tout):
        blk = tin[...]
        u = lax.bitcast_convert_type(blk, jnp.uint32)
        r = (u + jnp.uint32(0x7FFF) + ((u >> 16) & jnp.uint32(1))) \
            & jnp.uint32(0xFFFF0000)
        w = (r[0:16, :] >> 16) | r[16:EMBED, :]
        w128 = jnp.concatenate(
            [w[:, q * _PACK_BA:(q + 1) * _PACK_BA] for q in range(8)],
            axis=0)
        wf = lax.bitcast_convert_type(w128, jnp.float32)
        tout[...] = lax.transpose(wf, (1, 0))

    return pl.pallas_call(
        body,
        grid=(_PACK_GRID,),
        in_specs=[pl.BlockSpec((EMBED, _PACK_BV), lambda i: (0, i))],
        out_specs=pl.BlockSpec((_PACK_BA, 128), lambda i: (i, 0)),
        out_shape=jax.ShapeDtypeStruct((_PACK_ROWS, 128), jnp.float32),
    )(tbl_t)


def _sc_pooled_classify(x2, table16, fw_pat, bias_pat):
    mesh = plsc.VectorSubcoreMesh(core_axis_name="c", subcore_axis_name="s")

    @functools.partial(
        pl.kernel,
        out_type=jax.ShapeDtypeStruct((BATCH // 32, 128), jnp.float32),
        mesh=mesh,
        scratch_types=[
            pltpu.VMEM((2, IDX_PER_CHUNK, 128), jnp.int32),
            pltpu.VMEM((2, CHUNK * SEQ, 16), jnp.float32),
            pltpu.VMEM((N_CHUNKS * VECS_PER_CHUNK // 8, 128), jnp.float32),
            pltpu.VMEM((EMBED, 16), jnp.float32),
            pltpu.VMEM((16,), jnp.float32),
            pltpu.VMEM((CHUNK * EMBED,), jnp.float32),
            pltpu.SemaphoreType.DMA,
            pltpu.SemaphoreType.DMA,
        ],
        compiler_params=pltpu.CompilerParams(
            needs_layout_passes=False, use_tc_tiling_on_sc=False),
    )
    def k(x2_hbm, tbl_hbm, fwp_hbm, fcb_hbm, out_hbm, xbuf, rows_v, outbuf,
          fwp_v, fcb_v, acc_v, sem0, sem1):
        wid = lax.axis_index("s") * NC + lax.axis_index("c")
        row0 = wid * ROWS_PER_W
        ixrow0 = wid * ROWS_PER_W * SEQ // 128
        sems = (sem0, sem1)

        pltpu.sync_copy(fwp_hbm, fwp_v)
        pltpu.sync_copy(fcb_hbm, fcb_v)
        inv_s = jnp.float32(1.0 / SEQ)
        bias_v = fcb_v[pl.ds(0, 16)]
        zero = jnp.zeros((16,), jnp.float32)
        lane = lax.iota(jnp.int32, 16)
        rbase = (lane >> 2) * EMBED
        himask = jnp.int32(-65536)

        def issue(ci, b):
            pltpu.sync_copy(
                x2_hbm.at[pl.ds(ixrow0 + ci * IDX_PER_CHUNK, IDX_PER_CHUNK), :],
                xbuf.at[b])
            for j in range(IDX_PER_CHUNK):
                pltpu.async_copy(
                    tbl_hbm.at[xbuf.at[b, j]],
                    rows_v.at[b, pl.ds(j * 128, 128), :],
                    sems[b],
                )

        def wait_buf(b):
            pltpu.make_async_copy(
                tbl_hbm.at[pl.ds(0, CHUNK * SEQ), :], rows_v.at[b], sems[b]
            ).wait()

        def compute(ci, b):
            for r in range(CHUNK):
                def inner(t, carry, r=r):
                    ps = list(carry)
                    base = r * SEQ + t * 8
                    for uu in range(8):
                        w = plsc.bitcast(rows_v[b, base + uu, pl.ds(0, 16)],
                                         jnp.int32)
                        lo = plsc.bitcast(w << 16, jnp.float32)
                        hi = plsc.bitcast(w & himask, jnp.float32)
                        j = (uu % 4) * 2
                        ps[j] = ps[j] + lo
                        ps[j + 1] = ps[j + 1] + hi
                    return tuple(ps)

                ps = lax.fori_loop(0, SEQ // 8, inner, (zero,) * 8)
                a0 = (ps[0] + ps[2]) + (ps[4] + ps[6])
                a1 = (ps[1] + ps[3]) + (ps[5] + ps[7])
                acc_v[pl.ds(r * EMBED, 16)] = a0
                acc_v[pl.ds(r * EMBED + 16, 16)] = a1

            for g in range(VECS_PER_CHUNK):
                res0 = zero
                res1 = zero
                idx0 = rbase + g * 4 * EMBED
                for d in range(0, EMBED, 2):
                    res0 = res0 + (plsc.load_gather(acc_v, [idx0 + d])
                                   * fwp_v[d, pl.ds(0, 16)])
                    res1 = res1 + (plsc.load_gather(acc_v, [idx0 + d + 1])
                                   * fwp_v[d + 1, pl.ds(0, 16)])
                res = (res0 + res1) * inv_s + bias_v
                kk = ci * VECS_PER_CHUNK + g
                outbuf[kk // 8, pl.ds((kk % 8) * 16, 16)] = res

        issue(0, 0)

        def body2(h, _):
            for p in range(2):
                ci = h * 2 + p
                nxt = lax.rem(ci + 1, N_CHUNKS)
                issue(nxt, 1 - p)
                wait_buf(p)
                compute(ci, p)
            return ()

        lax.fori_loop(0, N_CHUNKS // 2, body2, ())
        wait_buf(0)

        n_out = N_CHUNKS * VECS_PER_CHUNK // 8
        pltpu.sync_copy(outbuf, out_hbm.at[pl.ds(wid * n_out, n_out), :])

    return k(x2, table16, fw_pat, bias_pat)


def kernel(x, table, fc_w, fc_b):
    y = x.astype(jnp.int32)
    x2 = ((y & jnp.int32(-8192)) | ((y & 1023) << 3) | ((y >> 10) & 7))
    x2 = x2.reshape(BATCH * SEQ // 128, 128)
    tbl_pk = _tc_pack_bf16(table.T)
    table16 = tbl_pk.reshape(_PACK_ROWS * 8, 16)
    fw_pat = jnp.tile(fc_w.T.astype(jnp.float32), (1, 4))
    bias_pat = jnp.tile(fc_b.astype(jnp.float32), 4)
    out = _sc_pooled_classify(x2, table16, fw_pat, bias_pat)
    return out.reshape(BATCH, NUM_CLASSES)
